# baseline (speedup 1.0000x reference)
import functools

import jax
import jax.numpy as jnp
from jax import lax
from jax.experimental import pallas as pl
from jax.experimental.pallas import tpu as pltpu
from jax.experimental.pallas import tpu_sc as plsc

_NC, _NS = 2, 16
_NW = _NC * _NS


def _make_table_kernel(V, H):
    VQ = V * H // 128
    CQ = 128
    CIN = CQ * 4
    NFULL = VQ // CQ
    RQ = VQ - NFULL * CQ
    RIN = RQ * 4
    KMAX = -(-NFULL // _NW)
    NSA = 3
    KTRI = -(-KMAX // NSA)

    mesh = plsc.VectorSubcoreMesh(core_axis_name="c", subcore_axis_name="s")

    @functools.partial(
        pl.kernel,
        mesh=mesh,
        compiler_params=pltpu.CompilerParams(
            use_tc_tiling_on_sc=True, needs_layout_passes=False
        ),
        out_type=jax.ShapeDtypeStruct((VQ, 128), jnp.float32),
        scratch_types=[
            [pltpu.VMEM((H, CIN), jnp.float32) for _ in range(NSA)],
            [pltpu.VMEM((CQ, 128), jnp.float32) for _ in range(NSA)],
            pltpu.VMEM((RIN, H), jnp.float32),
            [pltpu.SemaphoreType.DMA for _ in range(NSA)],
            [pltpu.SemaphoreType.DMA for _ in range(NSA)],
        ],
    )
    def ka(embt_hbm, tail_hbm, embq_hbm, vins, vouts, vtail, sins, souts):
        wid = lax.axis_index("s") * _NC + lax.axis_index("c")

        iotas = [lax.iota(jnp.int32, 16) + (l0 % 32) for l0 in range(0, 128, 16)]

        def chunk_of(kk):
            return kk * _NW + wid

        def start_in(kk, slot):
            c = chunk_of(kk)

            @pl.when(c < NFULL)
            def _():
                cin0 = pl.multiple_of(c * CIN, CIN)
                for tr in range(H // 8):
                    pltpu.async_copy(
                        embt_hbm.at[pl.ds(tr * 8, 8), pl.ds(cin0, CIN)],
                        vins[slot].at[pl.ds(tr * 8, 8)],
                        sins[slot],
                    )

        def wait_in(kk, slot):
            @pl.when(chunk_of(kk) < NFULL)
            def _():
                pltpu.make_async_copy(
                    embt_hbm.at[:, pl.ds(0, CIN)], vins[slot], sins[slot]
                ).wait()

        def wait_out(kk, slot):
            @pl.when(chunk_of(kk) < NFULL)
            def _():
                pltpu.make_async_copy(
                    vouts[slot], embq_hbm.at[pl.ds(0, CQ)], souts[slot]
                ).wait()

        def compute_and_write(kk, slot):
            c = chunk_of(kk)

            @pl.when(c < NFULL)
            def _():
                vbuf = vins[slot]
                obuf = vouts[slot]

                def row_body(r, carry):
                    c4 = r * 4
                    for j, l0 in enumerate(range(0, 128, 16)):
                        cols = jnp.full((16,), c4 + l0 // 32, jnp.int32)
                        vals = plsc.load_gather(vbuf, [iotas[j], cols])
                        obuf[r, pl.ds(l0, 16)] = vals
                    return carry

                lax.fori_loop(0, CQ, row_body, 0, unroll=8)
                q0 = pl.multiple_of(c * CQ, 8)
                pltpu.async_copy(obuf, embq_hbm.at[pl.ds(q0, CQ)], souts[slot])

        for j in range(NSA - 1):
            start_in(j, j)

        def tri_body(p, carry):
            for i in range(NSA):
                kk = p * NSA + i
                start_in(kk + NSA - 1, (i + NSA - 1) % NSA)
                wait_in(kk, i)

                @pl.when(kk >= NSA)
                def _():
                    wait_out(kk - NSA, i)

                compute_and_write(kk, i)
            return carry

        lax.fori_loop(0, KTRI, tri_body, 0)
        for j in range(NSA):
            kk = KTRI * NSA - NSA + j
            wait_out(kk, kk % NSA)

        if RQ:
            @pl.when(wid == 0)
            def _():
                pltpu.sync_copy(tail_hbm, vtail)
                for r in range(RQ):
                    for j, l0 in enumerate(range(0, 128, 16)):
                        rows = jnp.full((16,), r * 4 + l0 // 32, jnp.int32)
                        vals = plsc.load_gather(vtail, [rows, iotas[j]])
                        vouts[0][r, pl.ds(l0, 16)] = vals
                pltpu.sync_copy(
                    vouts[0].at[pl.ds(0, RQ)],
                    embq_hbm.at[pl.ds(NFULL * CQ, RQ)],
                )

    return ka


def _make_lookup_kernel(B, S, H, V):
    BW = B // _NW
    HT = H // 8
    OROWS = S * HT * (B // BW) * 8
    NSLOT = 8
    assert BW == 128 and H == 32 and S % NSLOT == 0

    mesh = plsc.VectorSubcoreMesh(core_axis_name="c", subcore_axis_name="s")

    @functools.partial(
        pl.kernel,
        mesh=mesh,
        compiler_params=pltpu.CompilerParams(
            use_tc_tiling_on_sc=False, needs_layout_passes=False
        ),
        out_type=jax.ShapeDtypeStruct((S * HT, B // BW, 8, 128), jnp.float32),
        scratch_types=[
            pltpu.VMEM((S, BW), jnp.int32),
            [pltpu.VMEM((BW, H), jnp.float32) for _ in range(NSLOT)],
            [pltpu.VMEM((HT, 8, BW), jnp.float32) for _ in range(NSLOT)],
            pltpu.VMEM((S, H), jnp.float32),
            [pltpu.SemaphoreType.DMA for _ in range(NSLOT)],
            [pltpu.SemaphoreType.DMA for _ in range(NSLOT)],
        ],
    )
    def kb(xt_hbm, emb_hbm, pos_hbm, out_hbm, idx_v, gbs, obs, pos_v,
           gsems, wsems):
        wid = lax.axis_index("s") * _NC + lax.axis_index("c")
        bw0 = pl.multiple_of(wid * BW, BW)
        pltpu.sync_copy(xt_hbm.at[:, pl.ds(bw0, BW)], idx_v)
        pltpu.sync_copy(pos_hbm, pos_v)

        iota16 = lax.iota(jnp.int32, 16)

        def start_gather(s, slot):
            pltpu.async_copy(emb_hbm.at[idx_v.at[s]], gbs[slot], gsems[slot])

        def extract_block(s, slot):
            gb = gbs[slot]
            ob = obs[slot]
            srows = jnp.full((16,), s, jnp.int32)

            def h_body(h, carry):
                cols = jnp.full((16,), h, jnp.int32)
                pv = plsc.load_gather(pos_v, [srows, cols])
                ht = lax.div(h, 8)
                hs = lax.rem(h, 8)
                for g in range(BW // 16):
                    rows = iota16 + (g * 16)
                    vals = plsc.load_gather(gb, [rows, cols])
                    ob[ht, hs, pl.ds(g * 16, 16)] = vals + pv
                return carry

            lax.fori_loop(0, H, h_body, 0, unroll=8)

        def start_write(s, slot):
            pltpu.async_copy(
                obs[slot],
                out_hbm.at[pl.ds(pl.multiple_of(s * HT, HT), HT), wid],
                wsems[slot],
            )

        def wait_write(slot):
            pltpu.make_async_copy(
                obs[slot], out_hbm.at[pl.ds(0, HT), 0], wsems[slot]
            ).wait()

        def wait_gather(slot):
            pltpu.make_async_copy(
                emb_hbm.at[pl.ds(0, BW)], gbs[slot], gsems[slot]
            ).wait()

        for s0 in range(NSLOT - 1):
            start_gather(s0, s0)

        def quad_body(p, carry):
            for i in range(NSLOT):
                s = p * NSLOT + i

                @pl.when(s + NSLOT - 1 < S)
                def _():
                    start_gather(s + NSLOT - 1, (i + NSLOT - 1) % NSLOT)

                wait_gather(i)

                @pl.when(s >= NSLOT)
                def _():
                    wait_write(i)

                extract_block(s, i)
                start_write(s, i)
            return carry

        lax.fori_loop(0, S // NSLOT, quad_body, 0)
        for slot in range(NSLOT):
            wait_write(slot)

    return kb


def kernel(x, emb, pos_emb):
    B, S = x.shape
    V, H = emb.shape
    ka = _make_table_kernel(V, H)
    kb = _make_lookup_kernel(B, S, H, V)
    embq = ka(emb.T, emb[V - (V % 512):])
    embl = embq.reshape(V, H)
    xt = x.T.astype(jnp.int32)
    ob2 = kb(xt, embl, pos_emb)
    o5 = ob2.reshape(S, H // 8, B // 128, 8, 128)
    return o5.transpose(2, 4, 0, 1, 3).reshape(B, S, H)


# speedup vs baseline: 1.5853x; 1.5853x over previous
import functools

import jax
import jax.numpy as jnp
from jax import lax
from jax.experimental import pallas as pl
from jax.experimental.pallas import tpu as pltpu
from jax.experimental.pallas import tpu_sc as plsc

_NC, _NS = 2, 16
_NW = _NC * _NS


def _make_table_kernel(V, H):
    VQ = V * H // 128
    CQ = 128
    CIN = CQ * 4
    NFULL = VQ // CQ
    RQ = VQ - NFULL * CQ
    RIN = RQ * 4
    KMAX = -(-NFULL // _NW)
    NSA = 3
    KTRI = -(-KMAX // NSA)

    mesh = plsc.VectorSubcoreMesh(core_axis_name="c", subcore_axis_name="s")

    @functools.partial(
        pl.kernel,
        mesh=mesh,
        compiler_params=pltpu.CompilerParams(
            use_tc_tiling_on_sc=True, needs_layout_passes=False
        ),
        out_type=jax.ShapeDtypeStruct((VQ, 128), jnp.float32),
        scratch_types=[
            [pltpu.VMEM((H, CIN + 16 + 1), jnp.float32) for _ in range(NSA)],
            [pltpu.VMEM((CQ, 128), jnp.float32) for _ in range(NSA)],
            pltpu.VMEM((RIN, H), jnp.float32),
            [pltpu.SemaphoreType.DMA for _ in range(NSA)],
            [pltpu.SemaphoreType.DMA for _ in range(NSA)],
        ],
    )
    def ka(embt_hbm, tail_hbm, embq_hbm, vins, vouts, vtail, sins, souts):
        wid = lax.axis_index("s") * _NC + lax.axis_index("c")

        iotas = [lax.iota(jnp.int32, 16) + (l0 % 32) for l0 in range(0, 128, 16)]

        def chunk_of(kk):
            return kk * _NW + wid

        def start_in(kk, slot):
            c = chunk_of(kk)

            @pl.when(c < NFULL)
            def _():
                cin0 = pl.multiple_of(c * CIN, CIN)
                for tr in range(H // 8):
                    pltpu.async_copy(
                        embt_hbm.at[pl.ds(tr * 8, 8), pl.ds(cin0, CIN)],
                        vins[slot].at[pl.ds(tr * 8, 8), pl.ds(0, CIN)],
                        sins[slot],
                    )

        def wait_in(kk, slot):
            @pl.when(chunk_of(kk) < NFULL)
            def _():
                pltpu.make_async_copy(
                    embt_hbm.at[:, pl.ds(0, CIN)],
                    vins[slot].at[:, pl.ds(0, CIN)],
                    sins[slot],
                ).wait()

        def wait_out(kk, slot):
            @pl.when(chunk_of(kk) < NFULL)
            def _():
                pltpu.make_async_copy(
                    vouts[slot], embq_hbm.at[pl.ds(0, CQ)], souts[slot]
                ).wait()

        def compute_and_write(kk, slot):
            c = chunk_of(kk)

            @pl.when(c < NFULL)
            def _():
                vbuf = vins[slot]
                obuf = vouts[slot]

                def row_body(r, carry):
                    c4 = r * 4
                    for j, l0 in enumerate(range(0, 128, 16)):
                        cols = jnp.full((16,), c4 + l0 // 32, jnp.int32)
                        vals = plsc.load_gather(vbuf, [iotas[j], cols])
                        obuf[r, pl.ds(l0, 16)] = vals
                    return carry

                lax.fori_loop(0, CQ, row_body, 0, unroll=8)
                q0 = pl.multiple_of(c * CQ, 8)
                pltpu.async_copy(obuf, embq_hbm.at[pl.ds(q0, CQ)], souts[slot])

        for j in range(NSA - 1):
            start_in(j, j)

        def tri_body(p, carry):
            for i in range(NSA):
                kk = p * NSA + i
                start_in(kk + NSA - 1, (i + NSA - 1) % NSA)
                wait_in(kk, i)

                @pl.when(kk >= NSA)
                def _():
                    wait_out(kk - NSA, i)

                compute_and_write(kk, i)
            return carry

        lax.fori_loop(0, KTRI, tri_body, 0)
        for j in range(NSA):
            kk = KTRI * NSA - NSA + j
            wait_out(kk, kk % NSA)

        if RQ:
            @pl.when(wid == 0)
            def _():
                pltpu.sync_copy(tail_hbm, vtail)
                for r in range(RQ):
                    for j, l0 in enumerate(range(0, 128, 16)):
                        rows = jnp.full((16,), r * 4 + l0 // 32, jnp.int32)
                        vals = plsc.load_gather(vtail, [rows, iotas[j]])
                        vouts[0][r, pl.ds(l0, 16)] = vals
                pltpu.sync_copy(
                    vouts[0].at[pl.ds(0, RQ)],
                    embq_hbm.at[pl.ds(NFULL * CQ, RQ)],
                )

    return ka


def _make_lookup_kernel(B, S, H, V):
    BW = B // _NW
    HT = H // 8
    OROWS = S * HT * (B // BW) * 8
    NSLOT = 8
    assert BW == 128 and H == 32 and S % NSLOT == 0

    mesh = plsc.VectorSubcoreMesh(core_axis_name="c", subcore_axis_name="s")

    @functools.partial(
        pl.kernel,
        mesh=mesh,
        compiler_params=pltpu.CompilerParams(
            use_tc_tiling_on_sc=False, needs_layout_passes=False
        ),
        out_type=jax.ShapeDtypeStruct((OROWS, 128), jnp.float32),
        scratch_types=[
            pltpu.VMEM((S, BW), jnp.int32),
            [pltpu.VMEM((BW, H), jnp.float32) for _ in range(NSLOT)],
            [pltpu.VMEM((H, BW + 1), jnp.float32) for _ in range(NSLOT)],
            pltpu.VMEM((S, H), jnp.float32),
            [pltpu.SemaphoreType.DMA for _ in range(NSLOT)],
            [pltpu.SemaphoreType.DMA for _ in range(NSLOT)],
        ],
    )
    def kb(xt_hbm, emb_hbm, pos_hbm, out_hbm, idx_v, gbs, obs, pos_v,
           gsems, wsems):
        wid = lax.axis_index("s") * _NC + lax.axis_index("c")
        bw0 = pl.multiple_of(wid * BW, BW)
        pltpu.sync_copy(xt_hbm.at[:, pl.ds(bw0, BW)], idx_v)
        pltpu.sync_copy(pos_hbm, pos_v)

        iota16 = lax.iota(jnp.int32, 16)

        def start_gather(s, slot):
            pltpu.async_copy(emb_hbm.at[idx_v.at[s]], gbs[slot], gsems[slot])

        def extract_block(s, slot):
            gb = gbs[slot]
            ob = obs[slot]
            pv0 = pos_v[s, pl.ds(0, 16)]
            pv1 = pos_v[s, pl.ds(16, 16)]
            rows1 = iota16 + 16

            def j_body(j, carry):
                cols = jnp.full((16,), j, jnp.int32)
                v0 = gb[j, pl.ds(0, 16)] + pv0
                v1 = gb[j, pl.ds(16, 16)] + pv1
                plsc.store_scatter(ob, [iota16, cols], v0)
                plsc.store_scatter(ob, [rows1, cols], v1)
                return carry

            lax.fori_loop(0, BW, j_body, 0, unroll=4)

        def start_write(s, slot):
            for ht in range(HT):
                r0 = ((s * HT + ht) * (B // BW) + wid) * 8
                pltpu.async_copy(
                    obs[slot].at[pl.ds(ht * 8, 8), pl.ds(0, 128)],
                    out_hbm.at[pl.ds(pl.multiple_of(r0, 8), 8)],
                    wsems[slot],
                )

        def wait_write(slot):
            pltpu.make_async_copy(
                obs[slot].at[:, pl.ds(0, 128)],
                out_hbm.at[pl.ds(0, H)],
                wsems[slot],
            ).wait()

        def wait_gather(slot):
            pltpu.make_async_copy(
                emb_hbm.at[pl.ds(0, BW)], gbs[slot], gsems[slot]
            ).wait()

        for s0 in range(NSLOT - 1):
            start_gather(s0, s0)

        def quad_body(p, carry):
            for i in range(NSLOT):
                s = p * NSLOT + i

                @pl.when(s + NSLOT - 1 < S)
                def _():
                    start_gather(s + NSLOT - 1, (i + NSLOT - 1) % NSLOT)

                wait_gather(i)

                @pl.when(s >= NSLOT)
                def _():
                    wait_write(i)

                extract_block(s, i)
                start_write(s, i)
            return carry

        lax.fori_loop(0, S // NSLOT, quad_body, 0)
        for slot in range(NSLOT):
            wait_write(slot)

    return kb


def kernel(x, emb, pos_emb):
    B, S = x.shape
    V, H = emb.shape
    ka = _make_table_kernel(V, H)
    kb = _make_lookup_kernel(B, S, H, V)
    embq = ka(emb.T, emb[V - (V % 512):])
    embl = embq.reshape(V, H)
    xt = x.T.astype(jnp.int32)
    ob2 = kb(xt, embl, pos_emb)
    o5 = ob2.reshape(S, H // 8, B // 128, 8, 128)
    return o5.transpose(2, 4, 0, 1, 3).reshape(B, S, H)


# speedup vs baseline: 2.0251x; 1.2774x over previous
import functools

import jax
import jax.numpy as jnp
from jax import lax
from jax.experimental import pallas as pl
from jax.experimental.pallas import tpu as pltpu
from jax.experimental.pallas import tpu_sc as plsc

_NC, _NS = 2, 16
_NW = _NC * _NS


def _make_table_kernel(V, H):
    VQ = V * H // 128
    CQ = 128
    CIN = CQ * 4
    NFULL = VQ // CQ
    RQ = VQ - NFULL * CQ
    RIN = RQ * 4
    KMAX = -(-NFULL // _NW)
    NSA = 3
    KTRI = -(-KMAX // NSA)

    mesh = plsc.VectorSubcoreMesh(core_axis_name="c", subcore_axis_name="s")

    @functools.partial(
        pl.kernel,
        mesh=mesh,
        compiler_params=pltpu.CompilerParams(
            use_tc_tiling_on_sc=True, needs_layout_passes=False
        ),
        out_type=jax.ShapeDtypeStruct((VQ, 128), jnp.float32),
        scratch_types=[
            [pltpu.VMEM((H, CIN + 16 + 1), jnp.float32) for _ in range(NSA)],
            [pltpu.VMEM((CQ, 128), jnp.float32) for _ in range(NSA)],
            pltpu.VMEM((RIN, H), jnp.float32),
            [pltpu.SemaphoreType.DMA for _ in range(NSA)],
            [pltpu.SemaphoreType.DMA for _ in range(NSA)],
        ],
    )
    def ka(embt_hbm, tail_hbm, embq_hbm, vins, vouts, vtail, sins, souts):
        wid = lax.axis_index("s") * _NC + lax.axis_index("c")

        iotas = [lax.iota(jnp.int32, 16) + (l0 % 32) for l0 in range(0, 128, 16)]

        def chunk_of(kk):
            return kk * _NW + wid

        def start_in(kk, slot):
            c = chunk_of(kk)

            @pl.when(c < NFULL)
            def _():
                cin0 = pl.multiple_of(c * CIN, CIN)
                for tr in range(H // 8):
                    pltpu.async_copy(
                        embt_hbm.at[pl.ds(tr * 8, 8), pl.ds(cin0, CIN)],
                        vins[slot].at[pl.ds(tr * 8, 8), pl.ds(0, CIN)],
                        sins[slot],
                    )

        def wait_in(kk, slot):
            @pl.when(chunk_of(kk) < NFULL)
            def _():
                pltpu.make_async_copy(
                    embt_hbm.at[:, pl.ds(0, CIN)],
                    vins[slot].at[:, pl.ds(0, CIN)],
                    sins[slot],
                ).wait()

        def wait_out(kk, slot):
            @pl.when(chunk_of(kk) < NFULL)
            def _():
                pltpu.make_async_copy(
                    vouts[slot], embq_hbm.at[pl.ds(0, CQ)], souts[slot]
                ).wait()

        def compute_and_write(kk, slot):
            c = chunk_of(kk)

            @pl.when(c < NFULL)
            def _():
                vbuf = vins[slot]
                obuf = vouts[slot]

                def row_body(r, carry):
                    c4 = r * 4
                    for j, l0 in enumerate(range(0, 128, 16)):
                        cols = jnp.full((16,), c4 + l0 // 32, jnp.int32)
                        vals = plsc.load_gather(vbuf, [iotas[j], cols])
                        obuf[r, pl.ds(l0, 16)] = vals
                    return carry

                lax.fori_loop(0, CQ, row_body, 0, unroll=8)
                q0 = pl.multiple_of(c * CQ, 8)
                pltpu.async_copy(obuf, embq_hbm.at[pl.ds(q0, CQ)], souts[slot])

        for j in range(NSA - 1):
            start_in(j, j)

        def tri_body(p, carry):
            for i in range(NSA):
                kk = p * NSA + i
                start_in(kk + NSA - 1, (i + NSA - 1) % NSA)
                wait_in(kk, i)

                @pl.when(kk >= NSA)
                def _():
                    wait_out(kk - NSA, i)

                compute_and_write(kk, i)
            return carry

        lax.fori_loop(0, KTRI, tri_body, 0)
        for j in range(NSA):
            kk = KTRI * NSA - NSA + j
            wait_out(kk, kk % NSA)

        if RQ:
            @pl.when(wid == 0)
            def _():
                pltpu.sync_copy(tail_hbm, vtail)
                for r in range(RQ):
                    for j, l0 in enumerate(range(0, 128, 16)):
                        rows = jnp.full((16,), r * 4 + l0 // 32, jnp.int32)
                        vals = plsc.load_gather(vtail, [rows, iotas[j]])
                        vouts[0][r, pl.ds(l0, 16)] = vals
                pltpu.sync_copy(
                    vouts[0].at[pl.ds(0, RQ)],
                    embq_hbm.at[pl.ds(NFULL * CQ, RQ)],
                )

    return ka


def _make_lookup_kernel(B, S, H, V):
    BW = B // _NW
    HT = H // 8
    OROWS = S * HT * (B // BW) * 8
    NSLOT = 8
    assert BW == 128 and H == 32 and S % NSLOT == 0

    mesh = plsc.VectorSubcoreMesh(core_axis_name="c", subcore_axis_name="s")

    @functools.partial(
        pl.kernel,
        mesh=mesh,
        compiler_params=pltpu.CompilerParams(
            use_tc_tiling_on_sc=False, needs_layout_passes=False
        ),
        out_type=jax.ShapeDtypeStruct((OROWS, 128), jnp.float32),
        scratch_types=[
            pltpu.VMEM((S, BW), jnp.int32),
            [pltpu.VMEM((BW, H), jnp.float32) for _ in range(NSLOT)],
            [pltpu.VMEM((H, BW + 1), jnp.float32) for _ in range(NSLOT)],
            pltpu.VMEM((S, H), jnp.float32),
            [pltpu.SemaphoreType.DMA for _ in range(NSLOT)],
            [pltpu.SemaphoreType.DMA for _ in range(NSLOT)],
        ],
    )
    def kb(xt_hbm, emb_hbm, pos_hbm, out_hbm, idx_v, gbs, obs, pos_v,
           gsems, wsems):
        wid = lax.axis_index("s") * _NC + lax.axis_index("c")
        bw0 = pl.multiple_of(wid * BW, BW)
        pltpu.sync_copy(xt_hbm.at[:, pl.ds(bw0, BW)], idx_v)
        pltpu.sync_copy(pos_hbm, pos_v)

        iota16 = lax.iota(jnp.int32, 16)

        def start_gather(s, slot):
            pltpu.async_copy(emb_hbm.at[idx_v.at[s]], gbs[slot], gsems[slot])

        def extract_block(s, slot):
            gb = gbs[slot]
            ob = obs[slot]
            pv0 = pos_v[s, pl.ds(0, 16)]
            pv1 = pos_v[s, pl.ds(16, 16)]
            rows1 = iota16 + 16

            def j_body(j, carry):
                cols = jnp.full((16,), j, jnp.int32)
                v0 = gb[j, pl.ds(0, 16)] + pv0
                v1 = gb[j, pl.ds(16, 16)] + pv1
                plsc.store_scatter(ob, [iota16, cols], v0)
                plsc.store_scatter(ob, [rows1, cols], v1)
                return carry

            lax.fori_loop(0, BW, j_body, 0, unroll=4)

        def start_write(s, slot):
            for ht in range(HT):
                r0 = ((s * HT + ht) * (B // BW) + wid) * 8
                pltpu.async_copy(
                    obs[slot].at[pl.ds(ht * 8, 8), pl.ds(0, 128)],
                    out_hbm.at[pl.ds(pl.multiple_of(r0, 8), 8)],
                    wsems[slot],
                )

        def wait_write(slot):
            pltpu.make_async_copy(
                obs[slot].at[:, pl.ds(0, 128)],
                out_hbm.at[pl.ds(0, H)],
                wsems[slot],
            ).wait()

        def wait_gather(slot):
            pltpu.make_async_copy(
                emb_hbm.at[pl.ds(0, BW)], gbs[slot], gsems[slot]
            ).wait()

        for s0 in range(NSLOT - 1):
            start_gather(s0, s0)

        def quad_body(p, carry):
            for i in range(NSLOT):
                s = p * NSLOT + i

                @pl.when(s + NSLOT - 1 < S)
                def _():
                    start_gather(s + NSLOT - 1, (i + NSLOT - 1) % NSLOT)

                wait_gather(i)

                @pl.when(s >= NSLOT)
                def _():
                    wait_write(i)

                extract_block(s, i)
                start_write(s, i)
            return carry

        lax.fori_loop(0, S // NSLOT, quad_body, 0)
        for slot in range(NSLOT):
            wait_write(slot)

    return kb


def kernel(x, emb, pos_emb):
    B, S = x.shape
    V, H = emb.shape
    kb = _make_lookup_kernel(B, S, H, V)
    xt = x.T.astype(jnp.int32)
    ob2 = kb(xt, emb, pos_emb)
    o5 = ob2.reshape(S, H // 8, B // 128, 8, 128)
    return o5.transpose(2, 4, 0, 1, 3).reshape(B, S, H)


# speedup vs baseline: 3.4229x; 1.6902x over previous
import functools

import jax
import jax.numpy as jnp
from jax import lax
from jax.experimental import pallas as pl
from jax.experimental.pallas import tpu as pltpu
from jax.experimental.pallas import tpu_sc as plsc

_NC, _NS = 2, 16
_NW = _NC * _NS


def _make_table_kernel(V, H):
    VQ = V * H // 128
    CQ = 128
    CIN = CQ * 4
    NFULL = VQ // CQ
    RQ = VQ - NFULL * CQ
    RIN = RQ * 4
    KMAX = -(-NFULL // _NW)
    NSA = 3
    KTRI = -(-KMAX // NSA)

    mesh = plsc.VectorSubcoreMesh(core_axis_name="c", subcore_axis_name="s")

    @functools.partial(
        pl.kernel,
        mesh=mesh,
        compiler_params=pltpu.CompilerParams(
            use_tc_tiling_on_sc=True, needs_layout_passes=False
        ),
        out_type=jax.ShapeDtypeStruct((VQ, 128), jnp.float32),
        scratch_types=[
            [pltpu.VMEM((H, CIN), jnp.float32) for _ in range(NSA)],
            [pltpu.VMEM((CQ, 128), jnp.float32) for _ in range(NSA)],
            pltpu.VMEM((RIN, H), jnp.float32),
            [pltpu.SemaphoreType.DMA for _ in range(NSA)],
            [pltpu.SemaphoreType.DMA for _ in range(NSA)],
        ],
    )
    def ka(embt_hbm, tail_hbm, embq_hbm, vins, vouts, vtail, sins, souts):
        wid = lax.axis_index("s") * _NC + lax.axis_index("c")

        iota16 = lax.iota(jnp.int32, 16)
        iotas = [iota16 + (l0 % 32) for l0 in range(0, 128, 16)]
        perms = [jnp.bitwise_and(iota16 + d, 15) for d in range(16)]

        def chunk_of(kk):
            return kk * _NW + wid

        def start_in(kk, slot):
            c = chunk_of(kk)

            @pl.when(c < NFULL)
            def _():
                cin0 = pl.multiple_of(c * CIN, CIN)
                for tr in range(H // 8):
                    pltpu.async_copy(
                        embt_hbm.at[pl.ds(tr * 8, 8), pl.ds(cin0, CIN)],
                        vins[slot].at[pl.ds(tr * 8, 8)],
                        sins[slot],
                    )

        def wait_in(kk, slot):
            @pl.when(chunk_of(kk) < NFULL)
            def _():
                pltpu.make_async_copy(
                    embt_hbm.at[:, pl.ds(0, CIN)], vins[slot], sins[slot]
                ).wait()

        def wait_out(kk, slot):
            @pl.when(chunk_of(kk) < NFULL)
            def _():
                pltpu.make_async_copy(
                    vouts[slot], embq_hbm.at[pl.ds(0, CQ)], souts[slot]
                ).wait()

        def compute_and_write(kk, slot):
            c = chunk_of(kk)

            @pl.when(c < NFULL)
            def _():
                vbuf = vins[slot]
                obuf = vouts[slot]

                def cb_body(cb, carry):
                    c0 = cb * 16
                    for h0 in (0, 16):
                        hrows = iota16 + h0
                        for d in range(16):
                            colsv = perms[d] + c0
                            vals = plsc.load_gather(vbuf, [hrows, colsv])
                            rowsq = lax.shift_right_logical(colsv, 2)
                            colsl = lax.shift_left(
                                jnp.bitwise_and(colsv, 3), 5) + hrows
                            plsc.store_scatter(obuf, [rowsq, colsl], vals)
                    return carry

                lax.fori_loop(0, CIN // 16, cb_body, 0)
                q0 = pl.multiple_of(c * CQ, 8)
                pltpu.async_copy(obuf, embq_hbm.at[pl.ds(q0, CQ)], souts[slot])

        for j in range(NSA - 1):
            start_in(j, j)

        def tri_body(p, carry):
            for i in range(NSA):
                kk = p * NSA + i
                start_in(kk + NSA - 1, (i + NSA - 1) % NSA)
                wait_in(kk, i)

                @pl.when(kk >= NSA)
                def _():
                    wait_out(kk - NSA, i)

                compute_and_write(kk, i)
            return carry

        lax.fori_loop(0, KTRI, tri_body, 0)
        for j in range(NSA):
            kk = KTRI * NSA - NSA + j
            wait_out(kk, kk % NSA)

        if RQ:
            @pl.when(wid == 0)
            def _():
                pltpu.sync_copy(tail_hbm, vtail)
                for r in range(RQ):
                    for j, l0 in enumerate(range(0, 128, 16)):
                        rows = jnp.full((16,), r * 4 + l0 // 32, jnp.int32)
                        vals = plsc.load_gather(vtail, [rows, iotas[j]])
                        vouts[0][r, pl.ds(l0, 16)] = vals
                pltpu.sync_copy(
                    vouts[0].at[pl.ds(0, RQ)],
                    embq_hbm.at[pl.ds(NFULL * CQ, RQ)],
                )

    return ka


def _make_lookup_kernel(B, S, H, V):
    BW = B // _NW
    HT = H // 8
    OROWS = S * HT * (B // BW) * 8
    NSLOT = 8
    assert BW == 128 and H == 32 and S % NSLOT == 0

    mesh = plsc.VectorSubcoreMesh(core_axis_name="c", subcore_axis_name="s")

    @functools.partial(
        pl.kernel,
        mesh=mesh,
        compiler_params=pltpu.CompilerParams(
            use_tc_tiling_on_sc=False, needs_layout_passes=False
        ),
        out_type=jax.ShapeDtypeStruct((OROWS, 128), jnp.float32),
        scratch_types=[
            pltpu.VMEM((S, BW), jnp.int32),
            [pltpu.VMEM((BW, H), jnp.float32) for _ in range(NSLOT)],
            [pltpu.VMEM((H, BW + 1), jnp.float32) for _ in range(NSLOT)],
            pltpu.VMEM((S, H), jnp.float32),
            [pltpu.SemaphoreType.DMA for _ in range(NSLOT)],
            [pltpu.SemaphoreType.DMA for _ in range(NSLOT)],
        ],
    )
    def kb(xt_hbm, emb_hbm, pos_hbm, out_hbm, idx_v, gbs, obs, pos_v,
           gsems, wsems):
        wid = lax.axis_index("s") * _NC + lax.axis_index("c")
        bw0 = pl.multiple_of(wid * BW, BW)
        pltpu.sync_copy(xt_hbm.at[:, pl.ds(bw0, BW)], idx_v)
        pltpu.sync_copy(pos_hbm, pos_v)

        iota16 = lax.iota(jnp.int32, 16)

        def start_gather(s, slot):
            pltpu.async_copy(emb_hbm.at[idx_v.at[s]], gbs[slot], gsems[slot])

        def extract_block(s, slot):
            gb = gbs[slot]
            ob = obs[slot]
            pv0 = pos_v[s, pl.ds(0, 16)]
            pv1 = pos_v[s, pl.ds(16, 16)]
            rows1 = iota16 + 16

            def j_body(j, carry):
                cols = jnp.full((16,), j, jnp.int32)
                v0 = gb[j, pl.ds(0, 16)] + pv0
                v1 = gb[j, pl.ds(16, 16)] + pv1
                plsc.store_scatter(ob, [iota16, cols], v0)
                plsc.store_scatter(ob, [rows1, cols], v1)
                return carry

            lax.fori_loop(0, BW, j_body, 0, unroll=4)

        def start_write(s, slot):
            for ht in range(HT):
                r0 = ((s * HT + ht) * (B // BW) + wid) * 8
                pltpu.async_copy(
                    obs[slot].at[pl.ds(ht * 8, 8), pl.ds(0, 128)],
                    out_hbm.at[pl.ds(pl.multiple_of(r0, 8), 8)],
                    wsems[slot],
                )

        def wait_write(slot):
            pltpu.make_async_copy(
                obs[slot].at[:, pl.ds(0, 128)],
                out_hbm.at[pl.ds(0, H)],
                wsems[slot],
            ).wait()

        def wait_gather(slot):
            pltpu.make_async_copy(
                emb_hbm.at[pl.ds(0, BW)], gbs[slot], gsems[slot]
            ).wait()

        for s0 in range(NSLOT - 1):
            start_gather(s0, s0)

        def quad_body(p, carry):
            for i in range(NSLOT):
                s = p * NSLOT + i

                @pl.when(s + NSLOT - 1 < S)
                def _():
                    start_gather(s + NSLOT - 1, (i + NSLOT - 1) % NSLOT)

                wait_gather(i)

                @pl.when(s >= NSLOT)
                def _():
                    wait_write(i)

                extract_block(s, i)
                start_write(s, i)
            return carry

        lax.fori_loop(0, S // NSLOT, quad_body, 0)
        for slot in range(NSLOT):
            wait_write(slot)

    return kb


def kernel(x, emb, pos_emb):
    B, S = x.shape
    V, H = emb.shape
    ka = _make_table_kernel(V, H)
    kb = _make_lookup_kernel(B, S, H, V)
    embq = ka(emb.T, emb[V - (V % 512):])
    embl = embq.reshape(V, H)
    xt = x.T.astype(jnp.int32)
    ob2 = kb(xt, embl, pos_emb)
    o5 = ob2.reshape(S, H // 8, B // 128, 8, 128)
    return o5.transpose(2, 4, 0, 1, 3).reshape(B, S, H)
